# baseline (speedup 1.0000x reference)
import functools

import jax
import jax.numpy as jnp
from jax import lax
from jax.experimental import pallas as pl
from jax.experimental.pallas import tpu as pltpu
from jax.experimental.pallas import tpu_sc as plsc

T = 4096
C = 768
E = 8
H = 768
BLK = 768
NB = 12
PT = NB * BLK

NC = 2
NS = 16
NW = NC * NS
RPW = T // NW


def _route_kernel(x_ref, w_ref, b_ref, dest_ref, be_ref, plan_ref, xpk_ref):
    lo_b = jax.lax.bitcast_convert_type(
        x_ref[:, :C // 2].astype(jnp.bfloat16).astype(jnp.float32), jnp.uint32)
    hi_b = jax.lax.bitcast_convert_type(
        x_ref[:, C // 2:].astype(jnp.bfloat16).astype(jnp.float32), jnp.uint32)
    pk = jax.lax.shift_right_logical(lo_b, jnp.uint32(16)) | (
        hi_b & jnp.uint32(0xFFFF0000))
    xpk_ref[...] = jax.lax.bitcast_convert_type(pk, jnp.int32)

    logits = jnp.dot(x_ref[...], w_ref[...],
                     preferred_element_type=jnp.float32) + b_ref[...]
    m = jnp.max(logits, axis=1, keepdims=True)
    eio = lax.broadcasted_iota(jnp.int32, (T, E), 1)
    routes = jnp.min(jnp.where(logits >= m, eio, E), axis=1, keepdims=True)
    onehot = (eio == routes).astype(jnp.float32)

    a = onehot
    k = 1
    while k < T:
        a = a + jnp.concatenate(
            [jnp.zeros((k, E), jnp.float32), a[:T - k]], axis=0)
        k *= 2
    counts = a[T - 1:T, :]
    nb = jnp.ceil(counts / BLK)

    c = nb
    k = 1
    while k < E:
        c = c + jnp.concatenate(
            [jnp.zeros((1, k), jnp.float32), c[:, :E - k]], axis=1)
        k *= 2
    start = c - nb

    dest_f = jnp.sum(onehot * (start * BLK + a - 1.0), axis=1, keepdims=True)
    dest_ref[...] = dest_f.astype(jnp.int32)

    nvf = c[:, E - 1:E]
    nvi = nvf.astype(jnp.int32)
    bio = lax.broadcasted_iota(jnp.int32, (NB, E), 0)
    bclamp = jnp.minimum(bio, nvi - 1)
    eio2 = lax.broadcasted_iota(jnp.int32, (NB, E), 1)
    pred = jnp.logical_and(bclamp >= start.astype(jnp.int32), counts > 0.5)
    bei = jnp.max(jnp.where(pred, eio2, -1), axis=1, keepdims=True)
    be_ref[...] = bei

    bef = bei.astype(jnp.float32)
    prevf = jnp.concatenate(
        [jnp.full((1, 1), -1.0, jnp.float32), bef[:NB - 1]], axis=0)
    firstf = (bef != prevf).astype(jnp.float32)
    ridf = firstf
    k = 1
    while k < NB:
        ridf = ridf + jnp.concatenate(
            [jnp.zeros((k, 1), jnp.float32), ridf[:NB - k]], axis=0)
        k *= 2
    ridf = ridf - 1.0

    io0 = lax.broadcasted_iota(jnp.int32, (NB, NB), 0).astype(jnp.float32)
    io1 = lax.broadcasted_iota(jnp.int32, (NB, NB), 1).astype(jnp.float32)
    eye = (io0 == io1).astype(jnp.float32)
    ones_row = jnp.ones((1, NB), jnp.float32)

    def to_row(v_col):
        return jnp.dot(ones_row, eye * v_col, preferred_element_type=jnp.float32)

    rid_row = to_row(ridf)
    first_row = to_row(firstf)
    be_row = to_row(bef)
    runm = (rid_row == io0).astype(jnp.float32) * first_row
    runs_col = jnp.sum(runm * (be_row + 1.0), axis=1, keepdims=True) - 1.0
    runs_row = to_row(runs_col)
    pfe = jnp.sum(((ridf + 2.0) == io1).astype(jnp.float32) * (runs_row + 1.0),
                  axis=1, keepdims=True) - 1.0
    er1 = jnp.sum((io1 == 1.0).astype(jnp.float32) * (runs_row + 1.0),
                  axis=1, keepdims=True) - 1.0
    er2 = jnp.sum((io1 == 2.0).astype(jnp.float32) * (runs_row + 1.0),
                  axis=1, keepdims=True) - 1.0
    slotf = ridf - 3.0 * jnp.floor(ridf / 3.0)
    pfsf = (ridf + 2.0) - 3.0 * jnp.floor((ridf + 2.0) / 3.0)
    blkio = lax.broadcasted_iota(jnp.int32, (NB, 1), 0)
    xsblk = jnp.minimum(blkio, nvi - 1).astype(jnp.float32)
    plan = jnp.concatenate(
        [bef, firstf, slotf, pfe, pfsf, er1, er2, xsblk],
        axis=1)
    plan_ref[...] = plan.astype(jnp.int32)


_route_call = pl.pallas_call(
    _route_kernel,
    out_shape=(
        jax.ShapeDtypeStruct((T, 1), jnp.int32),
        jax.ShapeDtypeStruct((NB, 1), jnp.int32),
        jax.ShapeDtypeStruct((NB, 8), jnp.int32),
        jax.ShapeDtypeStruct((T, C // 2), jnp.int32),
    ),
)


_sc_mesh = plsc.VectorSubcoreMesh(core_axis_name="c", subcore_axis_name="s")


@functools.partial(
    pl.kernel,
    mesh=_sc_mesh,
    out_type=jax.ShapeDtypeStruct((PT, C // 2), jnp.int32),
    scratch_types=[
        pltpu.VMEM((RPW,), jnp.int32),
        pltpu.VMEM((RPW, C // 2), jnp.int32),
        pltpu.SemaphoreType.DMA,
    ],
)
def _dispatch(x_hbm, dest_hbm, xs_hbm, idx_v, rows_v, sem):
    wid = lax.axis_index("s") * NC + lax.axis_index("c")
    base = wid * RPW
    pltpu.sync_copy(dest_hbm.at[pl.ds(base, RPW)], idx_v)
    pltpu.sync_copy(x_hbm.at[pl.ds(base, RPW)], rows_v)
    pltpu.async_copy(rows_v, xs_hbm.at[idx_v], sem).wait()


def _expert_kernel(be_ref, plan_ref, xs_ref, w1_hbm, b1_ref, w2_hbm, b2_ref,
                   ys_ref, w1v, w2v, s1, s2):
    i = pl.program_id(0)
    e = plan_ref[i, 0]
    first = plan_ref[i, 1]
    slot = plan_ref[i, 2]
    pfe = plan_ref[i, 3]
    pfs = plan_ref[i, 4]
    er1 = plan_ref[i, 5]
    er2 = plan_ref[i, 6]

    def start_copy(ex, sl):
        pltpu.make_async_copy(w1_hbm.at[ex], w1v.at[sl], s1.at[sl]).start()
        pltpu.make_async_copy(w2_hbm.at[ex], w2v.at[sl], s2.at[sl]).start()

    def wait_copy(sl):
        pltpu.make_async_copy(w1_hbm.at[0], w1v.at[sl], s1.at[sl]).wait()
        pltpu.make_async_copy(w2_hbm.at[0], w2v.at[sl], s2.at[sl]).wait()

    @pl.when(i == 0)
    def _():
        start_copy(e, 0)

    @pl.when(jnp.logical_and(i == 0, er1 >= 0))
    def _():
        start_copy(er1, 1)

    @pl.when(jnp.logical_and(i == 0, er2 >= 0))
    def _():
        start_copy(er2, 2)

    @pl.when(jnp.logical_and(i > 0, jnp.logical_and(first == 1, pfe >= 0)))
    def _():
        start_copy(pfe, pfs)

    @pl.when(first == 1)
    def _():
        wait_copy(slot)

    v = xs_ref[...]
    xlo = jax.lax.bitcast_convert_type(v << 16, jnp.float32)
    xhi = jax.lax.bitcast_convert_type(v & jnp.int32(-65536), jnp.float32)
    xsb = jnp.concatenate([xlo, xhi], axis=1).astype(jnp.bfloat16)
    h = jnp.dot(xsb,
                w1v[slot].astype(jnp.bfloat16),
                preferred_element_type=jnp.float32) + b1_ref[0]
    h = 0.5 * h * (1.0 + lax.erf(h * 0.7071067811865476))
    ys_ref[...] = jnp.dot(h.astype(jnp.bfloat16),
                          w2v[slot].astype(jnp.bfloat16),
                          preferred_element_type=jnp.float32) + b2_ref[0]


_EXPERT_IN_SPECS = [
    pl.BlockSpec((BLK, C // 2), lambda i, be, pln: (pln[i, 7], 0)),
    pl.BlockSpec(memory_space=pltpu.MemorySpace.HBM),
    pl.BlockSpec((1, 1, H), lambda i, be, pln: (be[i], 0, 0)),
    pl.BlockSpec(memory_space=pltpu.MemorySpace.HBM),
    pl.BlockSpec((1, 1, C), lambda i, be, pln: (be[i], 0, 0)),
]


_experts_call = pl.pallas_call(
    _expert_kernel,
    grid_spec=pltpu.PrefetchScalarGridSpec(
        num_scalar_prefetch=2,
        grid=(NB,),
        in_specs=_EXPERT_IN_SPECS,
        out_specs=pl.BlockSpec((BLK, C), lambda i, be, pln: (pln[i, 7], 0)),
        scratch_shapes=[
            pltpu.VMEM((3, C, H), jnp.float32),
            pltpu.VMEM((3, H, C), jnp.float32),
            pltpu.SemaphoreType.DMA((3,)),
            pltpu.SemaphoreType.DMA((3,)),
        ],
    ),
    out_shape=jax.ShapeDtypeStruct((PT, C), jnp.float32),
)


@functools.partial(
    pl.kernel,
    mesh=_sc_mesh,
    out_type=jax.ShapeDtypeStruct((T, C), jnp.float32),
    scratch_types=[
        pltpu.VMEM((RPW,), jnp.int32),
        pltpu.VMEM((RPW, C), jnp.float32),
        pltpu.SemaphoreType.DMA,
    ],
)
def _combine(ys_hbm, dest_hbm, out_hbm, idx_v, rows_v, sem):
    wid = lax.axis_index("s") * NC + lax.axis_index("c")
    base = wid * RPW
    pltpu.sync_copy(dest_hbm.at[pl.ds(base, RPW)], idx_v)
    pltpu.async_copy(ys_hbm.at[idx_v], rows_v, sem).wait()
    pltpu.sync_copy(rows_v, out_hbm.at[pl.ds(base, RPW)])


def kernel(x, switch_w, switch_b, w1, b1, w2, b2):
    Bx, Nx, Cx = x.shape
    xf = x.reshape(-1, Cx)
    dest2, be2, plan, xpk = _route_call(xf, switch_w, switch_b.reshape(1, E))
    dest = dest2.reshape(-1)
    block_expert = be2.reshape(-1)
    xs = _dispatch(xpk, dest)
    ys = _experts_call(block_expert, plan, xs, w1, b1.reshape(E, 1, H),
                       w2, b2.reshape(E, 1, C))
    outf = _combine(ys, dest)
    return outf.reshape(Bx, Nx, Cx)


# speedup vs baseline: 1.0105x; 1.0105x over previous
import functools

import jax
import jax.numpy as jnp
from jax import lax
from jax.experimental import pallas as pl
from jax.experimental.pallas import tpu as pltpu
from jax.experimental.pallas import tpu_sc as plsc

T = 4096
C = 768
E = 8
H = 768
BLK = 512
NB = 15
PT = NB * BLK

NC = 2
NS = 16
NW = NC * NS
RPW = T // NW


def _route_kernel(x_ref, w_ref, b_ref, dest_ref, be_ref, plan_ref, xpk_ref):
    lo_b = jax.lax.bitcast_convert_type(
        x_ref[:, :C // 2].astype(jnp.bfloat16).astype(jnp.float32), jnp.uint32)
    hi_b = jax.lax.bitcast_convert_type(
        x_ref[:, C // 2:].astype(jnp.bfloat16).astype(jnp.float32), jnp.uint32)
    pk = jax.lax.shift_right_logical(lo_b, jnp.uint32(16)) | (
        hi_b & jnp.uint32(0xFFFF0000))
    xpk_ref[...] = jax.lax.bitcast_convert_type(pk, jnp.int32)

    logits = jnp.dot(x_ref[...], w_ref[...],
                     preferred_element_type=jnp.float32) + b_ref[...]
    m = jnp.max(logits, axis=1, keepdims=True)
    eio = lax.broadcasted_iota(jnp.int32, (T, E), 1)
    routes = jnp.min(jnp.where(logits >= m, eio, E), axis=1, keepdims=True)
    onehot = (eio == routes).astype(jnp.float32)

    a = onehot
    k = 1
    while k < T:
        a = a + jnp.concatenate(
            [jnp.zeros((k, E), jnp.float32), a[:T - k]], axis=0)
        k *= 2
    counts = a[T - 1:T, :]
    nb = jnp.ceil(counts / BLK)

    c = nb
    k = 1
    while k < E:
        c = c + jnp.concatenate(
            [jnp.zeros((1, k), jnp.float32), c[:, :E - k]], axis=1)
        k *= 2
    start = c - nb

    dest_f = jnp.sum(onehot * (start * BLK + a - 1.0), axis=1, keepdims=True)
    dest_ref[...] = dest_f.astype(jnp.int32)

    nvf = c[:, E - 1:E]
    nvi = nvf.astype(jnp.int32)
    bio = lax.broadcasted_iota(jnp.int32, (NB, E), 0)
    bclamp = jnp.minimum(bio, nvi - 1)
    eio2 = lax.broadcasted_iota(jnp.int32, (NB, E), 1)
    pred = jnp.logical_and(bclamp >= start.astype(jnp.int32), counts > 0.5)
    bei = jnp.max(jnp.where(pred, eio2, -1), axis=1, keepdims=True)
    be_ref[...] = bei

    bef = bei.astype(jnp.float32)
    prevf = jnp.concatenate(
        [jnp.full((1, 1), -1.0, jnp.float32), bef[:NB - 1]], axis=0)
    firstf = (bef != prevf).astype(jnp.float32)
    ridf = firstf
    k = 1
    while k < NB:
        ridf = ridf + jnp.concatenate(
            [jnp.zeros((k, 1), jnp.float32), ridf[:NB - k]], axis=0)
        k *= 2
    ridf = ridf - 1.0

    io0 = lax.broadcasted_iota(jnp.int32, (NB, NB), 0).astype(jnp.float32)
    io1 = lax.broadcasted_iota(jnp.int32, (NB, NB), 1).astype(jnp.float32)
    eye = (io0 == io1).astype(jnp.float32)
    ones_row = jnp.ones((1, NB), jnp.float32)

    def to_row(v_col):
        return jnp.dot(ones_row, eye * v_col, preferred_element_type=jnp.float32)

    rid_row = to_row(ridf)
    first_row = to_row(firstf)
    be_row = to_row(bef)
    runm = (rid_row == io0).astype(jnp.float32) * first_row
    runs_col = jnp.sum(runm * (be_row + 1.0), axis=1, keepdims=True) - 1.0
    runs_row = to_row(runs_col)
    pfe = jnp.sum(((ridf + 2.0) == io1).astype(jnp.float32) * (runs_row + 1.0),
                  axis=1, keepdims=True) - 1.0
    er1 = jnp.sum((io1 == 1.0).astype(jnp.float32) * (runs_row + 1.0),
                  axis=1, keepdims=True) - 1.0
    er2 = jnp.sum((io1 == 2.0).astype(jnp.float32) * (runs_row + 1.0),
                  axis=1, keepdims=True) - 1.0
    slotf = ridf - 3.0 * jnp.floor(ridf / 3.0)
    pfsf = (ridf + 2.0) - 3.0 * jnp.floor((ridf + 2.0) / 3.0)
    blkio = lax.broadcasted_iota(jnp.int32, (NB, 1), 0)
    xsblk = jnp.minimum(blkio, nvi - 1).astype(jnp.float32)
    plan = jnp.concatenate(
        [bef, firstf, slotf, pfe, pfsf, er1, er2, xsblk],
        axis=1)
    plan_ref[...] = plan.astype(jnp.int32)


_route_call = pl.pallas_call(
    _route_kernel,
    out_shape=(
        jax.ShapeDtypeStruct((T, 1), jnp.int32),
        jax.ShapeDtypeStruct((NB, 1), jnp.int32),
        jax.ShapeDtypeStruct((NB, 8), jnp.int32),
        jax.ShapeDtypeStruct((T, C // 2), jnp.int32),
    ),
)


_sc_mesh = plsc.VectorSubcoreMesh(core_axis_name="c", subcore_axis_name="s")


@functools.partial(
    pl.kernel,
    mesh=_sc_mesh,
    out_type=jax.ShapeDtypeStruct((PT, C // 2), jnp.int32),
    scratch_types=[
        pltpu.VMEM((RPW,), jnp.int32),
        pltpu.VMEM((RPW, C // 2), jnp.int32),
        pltpu.SemaphoreType.DMA,
    ],
)
def _dispatch(x_hbm, dest_hbm, xs_hbm, idx_v, rows_v, sem):
    wid = lax.axis_index("s") * NC + lax.axis_index("c")
    base = wid * RPW
    pltpu.sync_copy(dest_hbm.at[pl.ds(base, RPW)], idx_v)
    pltpu.sync_copy(x_hbm.at[pl.ds(base, RPW)], rows_v)
    pltpu.async_copy(rows_v, xs_hbm.at[idx_v], sem).wait()


def _expert_kernel(be_ref, plan_ref, xs_ref, w1_hbm, b1_ref, w2_hbm, b2_ref,
                   ys_ref, w1v, w2v, s1, s2):
    i = pl.program_id(0)
    e = plan_ref[i, 0]
    first = plan_ref[i, 1]
    slot = plan_ref[i, 2]
    pfe = plan_ref[i, 3]
    pfs = plan_ref[i, 4]
    er1 = plan_ref[i, 5]
    er2 = plan_ref[i, 6]

    def start_copy(ex, sl):
        pltpu.make_async_copy(w1_hbm.at[ex], w1v.at[sl], s1.at[sl]).start()
        pltpu.make_async_copy(w2_hbm.at[ex], w2v.at[sl], s2.at[sl]).start()

    def wait_copy(sl):
        pltpu.make_async_copy(w1_hbm.at[0], w1v.at[sl], s1.at[sl]).wait()
        pltpu.make_async_copy(w2_hbm.at[0], w2v.at[sl], s2.at[sl]).wait()

    @pl.when(i == 0)
    def _():
        start_copy(e, 0)

    @pl.when(jnp.logical_and(i == 0, er1 >= 0))
    def _():
        start_copy(er1, 1)

    @pl.when(jnp.logical_and(i == 0, er2 >= 0))
    def _():
        start_copy(er2, 2)

    @pl.when(jnp.logical_and(i > 0, jnp.logical_and(first == 1, pfe >= 0)))
    def _():
        start_copy(pfe, pfs)

    @pl.when(first == 1)
    def _():
        wait_copy(slot)

    v = xs_ref[...]
    xlo = jax.lax.bitcast_convert_type(v << 16, jnp.float32)
    xhi = jax.lax.bitcast_convert_type(v & jnp.int32(-65536), jnp.float32)
    xsb = jnp.concatenate([xlo, xhi], axis=1).astype(jnp.bfloat16)
    h = jnp.dot(xsb,
                w1v[slot].astype(jnp.bfloat16),
                preferred_element_type=jnp.float32) + b1_ref[0]
    h = 0.5 * h * (1.0 + lax.erf(h * 0.7071067811865476))
    ys_ref[...] = jnp.dot(h.astype(jnp.bfloat16),
                          w2v[slot].astype(jnp.bfloat16),
                          preferred_element_type=jnp.float32) + b2_ref[0]


_EXPERT_IN_SPECS = [
    pl.BlockSpec((BLK, C // 2), lambda i, be, pln: (pln[i, 7], 0)),
    pl.BlockSpec(memory_space=pltpu.MemorySpace.HBM),
    pl.BlockSpec((1, 1, H), lambda i, be, pln: (be[i], 0, 0)),
    pl.BlockSpec(memory_space=pltpu.MemorySpace.HBM),
    pl.BlockSpec((1, 1, C), lambda i, be, pln: (be[i], 0, 0)),
]


_experts_call = pl.pallas_call(
    _expert_kernel,
    grid_spec=pltpu.PrefetchScalarGridSpec(
        num_scalar_prefetch=2,
        grid=(NB,),
        in_specs=_EXPERT_IN_SPECS,
        out_specs=pl.BlockSpec((BLK, C), lambda i, be, pln: (pln[i, 7], 0)),
        scratch_shapes=[
            pltpu.VMEM((3, C, H), jnp.float32),
            pltpu.VMEM((3, H, C), jnp.float32),
            pltpu.SemaphoreType.DMA((3,)),
            pltpu.SemaphoreType.DMA((3,)),
        ],
    ),
    out_shape=jax.ShapeDtypeStruct((PT, C), jnp.float32),
)


@functools.partial(
    pl.kernel,
    mesh=_sc_mesh,
    out_type=jax.ShapeDtypeStruct((T, C), jnp.float32),
    scratch_types=[
        pltpu.VMEM((RPW,), jnp.int32),
        pltpu.VMEM((RPW, C), jnp.float32),
        pltpu.SemaphoreType.DMA,
    ],
)
def _combine(ys_hbm, dest_hbm, out_hbm, idx_v, rows_v, sem):
    wid = lax.axis_index("s") * NC + lax.axis_index("c")
    base = wid * RPW
    pltpu.sync_copy(dest_hbm.at[pl.ds(base, RPW)], idx_v)
    pltpu.async_copy(ys_hbm.at[idx_v], rows_v, sem).wait()
    pltpu.sync_copy(rows_v, out_hbm.at[pl.ds(base, RPW)])


def kernel(x, switch_w, switch_b, w1, b1, w2, b2):
    Bx, Nx, Cx = x.shape
    xf = x.reshape(-1, Cx)
    dest2, be2, plan, xpk = _route_call(xf, switch_w, switch_b.reshape(1, E))
    dest = dest2.reshape(-1)
    block_expert = be2.reshape(-1)
    xs = _dispatch(xpk, dest)
    ys = _experts_call(block_expert, plan, xs, w1, b1.reshape(E, 1, H),
                       w2, b2.reshape(E, 1, C))
    outf = _combine(ys, dest)
    return outf.reshape(Bx, Nx, Cx)
